# baseline (speedup 1.0000x reference)
import jax
import jax.numpy as jnp
from jax.experimental import pallas as pl


def _body(cont_ref, cat_ref, w_ref, out_ref):
    n_fields, field_k, emb = w_ref.shape
    n_cont = cont_ref.shape[1]
    out_ref[:, 0:n_cont] = cont_ref[...]
    for i in range(n_fields):
        x = cat_ref[:, i * field_k:(i + 1) * field_k].astype(jnp.bfloat16)
        acc = jnp.dot(x, w_ref[i], preferred_element_type=jnp.float32)
        out_ref[:, n_cont + i * emb:n_cont + (i + 1) * emb] = acc


def kernel(continuous, category, W):
    B, n_cont = continuous.shape
    n_fields, field_k, emb = W.shape
    d_out = n_cont + n_fields * emb
    Bb = 128
    Wb = W.astype(jnp.bfloat16)
    return pl.pallas_call(
        _body,
        grid=(B // Bb,),
        in_specs=[
            pl.BlockSpec((Bb, n_cont), lambda b: (b, 0)),
            pl.BlockSpec((Bb, n_fields * field_k), lambda b: (b, 0)),
            pl.BlockSpec((n_fields, field_k, emb), lambda b: (0, 0, 0)),
        ],
        out_specs=pl.BlockSpec((Bb, d_out), lambda b: (b, 0)),
        out_shape=jax.ShapeDtypeStruct((B, d_out), jnp.float32),
    )(continuous, category, Wb)
